# baseline (speedup 1.0000x reference)
import functools

import jax
import jax.numpy as jnp
from jax import lax
from jax.experimental import pallas as pl
from jax.experimental.pallas import tpu as pltpu
from jax.experimental.pallas import tpu_sc as plsc

E = 8
K = 2
D = 1024
F = 2048
T = 2048
CAP = 640
TAU = 0.8
TK = T * K
S = E * CAP
NC, NS = 2, 16
NW = NC * NS
_BF = jnp.bfloat16


_TM = 256


def _route_plan_body(x_ref, mu_ref, ci_ref, g_ref, slot_ref, keep_ref,
                     map_ref, xbf_ref):
    xb = x_ref[...]
    ci = ci_ref[...].astype(_BF)
    xc = jnp.dot(xb.astype(_BF), ci, preferred_element_type=jnp.float32)
    x_term = jnp.sum(xc * xb, axis=1, keepdims=True)
    mu = mu_ref[...]
    muc = jnp.dot(mu.astype(_BF), ci, preferred_element_type=jnp.float32)
    mu_term = jnp.sum(muc * mu, axis=1)[None, :]
    cross = lax.dot_general(xc.astype(_BF), mu.astype(_BF),
                            (((1,), (1,)), ((), ())),
                            preferred_element_type=jnp.float32)
    dist = x_term - 2.0 * cross + mu_term
    logits = -dist / (TAU * jnp.sqrt(jnp.float32(D)))
    m = jnp.max(logits, axis=1, keepdims=True)
    ex = jnp.exp(logits - m)
    p = ex / jnp.sum(ex, axis=1, keepdims=True)
    lanes = lax.broadcasted_iota(jnp.int32, p.shape, 1)
    v1 = jnp.max(p, axis=1, keepdims=True)
    e0 = jnp.min(jnp.where(p == v1, lanes, E), axis=1, keepdims=True)
    p2 = jnp.where(lanes == e0, -jnp.inf, p)
    v2 = jnp.max(p2, axis=1, keepdims=True)
    e1 = jnp.min(jnp.where(p2 == v2, lanes, E), axis=1, keepdims=True)
    ssum = v1 + v2
    g_ref[...] = jnp.concatenate([v1 / ssum, v2 / ssum], axis=1)
    erow = lax.broadcasted_iota(jnp.int32, (T, E), 1)
    m0 = (e0 == erow).astype(jnp.float32)
    m1 = (e1 == erow).astype(jnp.float32)
    rr = lax.broadcasted_iota(jnp.int32, (T, T), 0)
    cc = lax.broadcasted_iota(jnp.int32, (T, T), 1)
    ltri = (cc <= rr).astype(_BF)
    c = jnp.dot(ltri, (m0 + m1).astype(_BF),
                preferred_element_type=jnp.float32)
    pos1 = jnp.sum(m1 * c, axis=1, keepdims=True) - 1.0
    pos0 = jnp.sum(m0 * (c - m1), axis=1, keepdims=True) - 1.0
    pos0i = pos0.astype(jnp.int32)
    pos1i = pos1.astype(jnp.int32)
    keep0 = pos0i < CAP
    keep1 = pos1i < CAP
    slot0 = e0 * CAP + jnp.minimum(pos0i, CAP - 1)
    slot1 = e1 * CAP + jnp.minimum(pos1i, CAP - 1)
    slot_ref[...] = jnp.concatenate([slot0, slot1], axis=1)
    keep_ref[...] = jnp.concatenate(
        [keep0.astype(jnp.int32), keep1.astype(jnp.int32)], axis=1)
    tokp1 = lax.broadcasted_iota(jnp.int32, (T, 1), 0) + 1
    thi = (tokp1 >> 6).astype(_BF)
    tlo = (tokp1 & 63).astype(_BF)
    lane_c = lax.broadcasted_iota(jnp.int32, (T, CAP), 1)
    a0 = ((pos0i == lane_c) & keep0).astype(_BF)
    a1 = ((pos1i == lane_c) & keep1).astype(_BF)
    m0b = m0.astype(_BF)
    m1b = m1.astype(_BF)
    dn = (((0,), (0,)), ((), ()))
    hi = (lax.dot_general(a0 * thi, m0b, dn, preferred_element_type=jnp.float32)
          + lax.dot_general(a1 * thi, m1b, dn,
                            preferred_element_type=jnp.float32))
    lo = (lax.dot_general(a0 * tlo, m0b, dn, preferred_element_type=jnp.float32)
          + lax.dot_general(a1 * tlo, m1b, dn,
                            preferred_element_type=jnp.float32))
    map_ref[...] = (64.0 * hi + lo).astype(jnp.int32) - 1
    xbf_ref[...] = xb.astype(_BF)


def _route_plan(x, mu, cov_inv):
    return pl.pallas_call(
        _route_plan_body,
        grid=(1,),
        in_specs=[
            pl.BlockSpec((T, D), lambda i: (0, 0)),
            pl.BlockSpec((E, D), lambda i: (0, 0)),
            pl.BlockSpec((D, D), lambda i: (0, 0)),
        ],
        out_specs=[
            pl.BlockSpec((T, K), lambda i: (0, 0)),
            pl.BlockSpec((T, K), lambda i: (0, 0)),
            pl.BlockSpec((T, K), lambda i: (0, 0)),
            pl.BlockSpec((CAP, E), lambda i: (0, 0)),
            pl.BlockSpec((T, D), lambda i: (0, 0)),
        ],
        out_shape=[
            jax.ShapeDtypeStruct((T, K), jnp.float32),
            jax.ShapeDtypeStruct((T, K), jnp.int32),
            jax.ShapeDtypeStruct((T, K), jnp.int32),
            jax.ShapeDtypeStruct((CAP, E), jnp.int32),
            jax.ShapeDtypeStruct((T, D), _BF),
        ],
    )(x, mu, cov_inv)


_NF = 2


def _moe_body(xbf_ref, mapc_ref, slot_ref, keep_ref, g_ref,
              w1_ref, b1_ref, w2_ref, b2_ref, y_ref, xe_s, o_s):
    e = pl.program_id(0)
    f = pl.program_id(1)

    @pl.when(f == 0)
    def _():
        mfull = mapc_ref[...]
        elane = lax.broadcasted_iota(jnp.int32, (CAP, E), 1)
        mcol = jnp.sum(jnp.where(elane == e, mfull, 0), axis=1,
                       keepdims=True)
        tlane = lax.broadcasted_iota(jnp.int32, (CAP, T), 1)
        pmat = (mcol == tlane).astype(_BF)
        xe = jnp.dot(pmat, xbf_ref[...], preferred_element_type=jnp.float32)
        xe_s[...] = xe.astype(_BF)

    h = jnp.dot(xe_s[...], w1_ref[0].astype(_BF),
                preferred_element_type=jnp.float32) + b1_ref[0]
    h = jnp.maximum(h, 0.0)
    ob = jnp.dot(h.astype(_BF), w2_ref[0].astype(_BF),
                 preferred_element_type=jnp.float32)

    @pl.when(f == 0)
    def _():
        o_s[...] = ob + b2_ref[0]

    @pl.when(f != 0)
    def _():
        o_s[...] += ob

    @pl.when(f == _NF - 1)
    def _():
        lane_c = lax.broadcasted_iota(jnp.int32, (T, CAP), 1)
        c0 = slot_ref[:, 0:1] - e * CAP
        c1 = slot_ref[:, 1:2] - e * CAP
        g = g_ref[...]
        cmat = (jnp.where((c0 == lane_c) & (keep_ref[:, 0:1] != 0),
                          g[:, 0:1], 0.0)
                + jnp.where((c1 == lane_c) & (keep_ref[:, 1:2] != 0),
                            g[:, 1:2], 0.0))
        contrib = jnp.dot(cmat.astype(_BF), o_s[...].astype(_BF),
                          preferred_element_type=jnp.float32)

        @pl.when(e == 0)
        def _():
            y_ref[...] = contrib

        @pl.when(e != 0)
        def _():
            y_ref[...] += contrib


def _moe(xbf, mapc, slot, keep, gates, W1, b1r, W2, b2r):
    return pl.pallas_call(
        _moe_body,
        grid=(E, _NF),
        in_specs=[
            pl.BlockSpec((T, D), lambda e, f: (0, 0)),
            pl.BlockSpec((CAP, E), lambda e, f: (0, 0)),
            pl.BlockSpec((T, K), lambda e, f: (0, 0)),
            pl.BlockSpec((T, K), lambda e, f: (0, 0)),
            pl.BlockSpec((T, K), lambda e, f: (0, 0)),
            pl.BlockSpec((1, D, F // _NF), lambda e, f: (e, 0, f)),
            pl.BlockSpec((1, 1, F // _NF), lambda e, f: (e, 0, f)),
            pl.BlockSpec((1, F // _NF, D), lambda e, f: (e, f, 0)),
            pl.BlockSpec((1, 1, D), lambda e, f: (e, 0, 0)),
        ],
        out_specs=pl.BlockSpec((T, D), lambda e, f: (0, 0)),
        out_shape=jax.ShapeDtypeStruct((T, D), jnp.float32),
        scratch_shapes=[
            pltpu.VMEM((CAP, D), _BF),
            pltpu.VMEM((CAP, D), jnp.float32),
        ],
    )(xbf, mapc, slot, keep, gates, W1, b1r, W2, b2r)


def kernel(x, mu, cov_inv, W1, b1, W2, b2):
    gates, slot, keep, mapc, xbf = _route_plan(x, mu, cov_inv)
    return _moe(xbf, mapc, slot, keep, gates, W1, b1.reshape(E, 1, F),
                W2, b2.reshape(E, 1, D))


# speedup vs baseline: 1.1332x; 1.1332x over previous
import functools

import jax
import jax.numpy as jnp
from jax import lax
from jax.experimental import pallas as pl
from jax.experimental.pallas import tpu as pltpu
from jax.experimental.pallas import tpu_sc as plsc

E = 8
K = 2
D = 1024
F = 2048
T = 2048
CAP = 640
TAU = 0.8
TK = T * K
S = E * CAP
NC, NS = 2, 16
NW = NC * NS
_BF = jnp.bfloat16


_TM = 256


def _route_plan_body(x_ref, mu_ref, ci_ref, g_ref, slot_ref, keep_ref,
                     map_ref, xbf_ref):
    xb = x_ref[...]
    ci = ci_ref[...].astype(_BF)
    xc = jnp.dot(xb.astype(_BF), ci, preferred_element_type=jnp.float32)
    x_term = jnp.sum(xc * xb, axis=1, keepdims=True)
    mu = mu_ref[...]
    muc = jnp.dot(mu.astype(_BF), ci, preferred_element_type=jnp.float32)
    mu_term = jnp.sum(muc * mu, axis=1)[None, :]
    cross = lax.dot_general(xc.astype(_BF), mu.astype(_BF),
                            (((1,), (1,)), ((), ())),
                            preferred_element_type=jnp.float32)
    dist = x_term - 2.0 * cross + mu_term
    logits = -dist / (TAU * jnp.sqrt(jnp.float32(D)))
    m = jnp.max(logits, axis=1, keepdims=True)
    ex = jnp.exp(logits - m)
    p = ex / jnp.sum(ex, axis=1, keepdims=True)
    lanes = lax.broadcasted_iota(jnp.int32, p.shape, 1)
    v1 = jnp.max(p, axis=1, keepdims=True)
    e0 = jnp.min(jnp.where(p == v1, lanes, E), axis=1, keepdims=True)
    p2 = jnp.where(lanes == e0, -jnp.inf, p)
    v2 = jnp.max(p2, axis=1, keepdims=True)
    e1 = jnp.min(jnp.where(p2 == v2, lanes, E), axis=1, keepdims=True)
    ssum = v1 + v2
    g_ref[...] = jnp.concatenate([v1 / ssum, v2 / ssum], axis=1)
    erow = lax.broadcasted_iota(jnp.int32, (T, E), 1)
    m0 = (e0 == erow).astype(jnp.float32)
    m1 = (e1 == erow).astype(jnp.float32)
    c = m0 + m1
    sh = 1
    while sh < T:
        c = c + jnp.concatenate(
            [jnp.zeros((sh, E), jnp.float32), c[: T - sh, :]], axis=0)
        sh *= 2
    pos1 = jnp.sum(m1 * c, axis=1, keepdims=True) - 1.0
    pos0 = jnp.sum(m0 * (c - m1), axis=1, keepdims=True) - 1.0
    pos0i = pos0.astype(jnp.int32)
    pos1i = pos1.astype(jnp.int32)
    keep0 = pos0i < CAP
    keep1 = pos1i < CAP
    slot0 = e0 * CAP + jnp.minimum(pos0i, CAP - 1)
    slot1 = e1 * CAP + jnp.minimum(pos1i, CAP - 1)
    slot_ref[...] = jnp.concatenate([slot0, slot1], axis=1)
    keep_ref[...] = jnp.concatenate(
        [keep0.astype(jnp.int32), keep1.astype(jnp.int32)], axis=1)
    tokp1 = lax.broadcasted_iota(jnp.int32, (T, 1), 0) + 1
    thi = (tokp1 >> 6).astype(_BF)
    tlo = (tokp1 & 63).astype(_BF)
    lane_c = lax.broadcasted_iota(jnp.int32, (T, CAP), 1)
    a0 = ((pos0i == lane_c) & keep0).astype(_BF)
    a1 = ((pos1i == lane_c) & keep1).astype(_BF)
    m0b = m0.astype(_BF)
    m1b = m1.astype(_BF)
    dn = (((0,), (0,)), ((), ()))
    hi = (lax.dot_general(a0 * thi, m0b, dn, preferred_element_type=jnp.float32)
          + lax.dot_general(a1 * thi, m1b, dn,
                            preferred_element_type=jnp.float32))
    lo = (lax.dot_general(a0 * tlo, m0b, dn, preferred_element_type=jnp.float32)
          + lax.dot_general(a1 * tlo, m1b, dn,
                            preferred_element_type=jnp.float32))
    map_ref[...] = (64.0 * hi + lo).astype(jnp.int32) - 1
    xbf_ref[...] = xb.astype(_BF)


def _route_plan(x, mu, cov_inv):
    return pl.pallas_call(
        _route_plan_body,
        grid=(1,),
        in_specs=[
            pl.BlockSpec((T, D), lambda i: (0, 0)),
            pl.BlockSpec((E, D), lambda i: (0, 0)),
            pl.BlockSpec((D, D), lambda i: (0, 0)),
        ],
        out_specs=[
            pl.BlockSpec((T, K), lambda i: (0, 0)),
            pl.BlockSpec((T, K), lambda i: (0, 0)),
            pl.BlockSpec((T, K), lambda i: (0, 0)),
            pl.BlockSpec((CAP, E), lambda i: (0, 0)),
            pl.BlockSpec((T, D), lambda i: (0, 0)),
        ],
        out_shape=[
            jax.ShapeDtypeStruct((T, K), jnp.float32),
            jax.ShapeDtypeStruct((T, K), jnp.int32),
            jax.ShapeDtypeStruct((T, K), jnp.int32),
            jax.ShapeDtypeStruct((CAP, E), jnp.int32),
            jax.ShapeDtypeStruct((T, D), _BF),
        ],
    )(x, mu, cov_inv)


_NF = 1


def _moe_body(xbf_ref, mapc_ref, slot_ref, keep_ref, g_ref,
              w1_ref, b1_ref, w2_ref, b2_ref, y_ref, xe_s, o_s):
    e = pl.program_id(0)
    f = pl.program_id(1)

    @pl.when(f == 0)
    def _():
        mfull = mapc_ref[...]
        elane = lax.broadcasted_iota(jnp.int32, (CAP, E), 1)
        mcol = jnp.sum(jnp.where(elane == e, mfull, 0), axis=1,
                       keepdims=True)
        tlane = lax.broadcasted_iota(jnp.int32, (CAP, T), 1)
        pmat = (mcol == tlane).astype(_BF)
        xe = jnp.dot(pmat, xbf_ref[...], preferred_element_type=jnp.float32)
        xe_s[...] = xe.astype(_BF)

    h = jnp.dot(xe_s[...], w1_ref[0].astype(_BF),
                preferred_element_type=jnp.float32) + b1_ref[0]
    h = jnp.maximum(h, 0.0)
    ob = jnp.dot(h.astype(_BF), w2_ref[0].astype(_BF),
                 preferred_element_type=jnp.float32)

    @pl.when(f == 0)
    def _():
        o_s[...] = ob + b2_ref[0]

    @pl.when(f != 0)
    def _():
        o_s[...] += ob

    @pl.when(f == _NF - 1)
    def _():
        lane_c = lax.broadcasted_iota(jnp.int32, (T, CAP), 1)
        c0 = slot_ref[:, 0:1] - e * CAP
        c1 = slot_ref[:, 1:2] - e * CAP
        g = g_ref[...]
        cmat = (jnp.where((c0 == lane_c) & (keep_ref[:, 0:1] != 0),
                          g[:, 0:1], 0.0)
                + jnp.where((c1 == lane_c) & (keep_ref[:, 1:2] != 0),
                            g[:, 1:2], 0.0))
        contrib = jnp.dot(cmat.astype(_BF), o_s[...].astype(_BF),
                          preferred_element_type=jnp.float32)

        @pl.when(e == 0)
        def _():
            y_ref[...] = contrib

        @pl.when(e != 0)
        def _():
            y_ref[...] += contrib


def _moe(xbf, mapc, slot, keep, gates, W1, b1r, W2, b2r):
    return pl.pallas_call(
        _moe_body,
        grid=(E, _NF),
        in_specs=[
            pl.BlockSpec((T, D), lambda e, f: (0, 0)),
            pl.BlockSpec((CAP, E), lambda e, f: (0, 0)),
            pl.BlockSpec((T, K), lambda e, f: (0, 0)),
            pl.BlockSpec((T, K), lambda e, f: (0, 0)),
            pl.BlockSpec((T, K), lambda e, f: (0, 0)),
            pl.BlockSpec((1, D, F // _NF), lambda e, f: (e, 0, f)),
            pl.BlockSpec((1, 1, F // _NF), lambda e, f: (e, 0, f)),
            pl.BlockSpec((1, F // _NF, D), lambda e, f: (e, f, 0)),
            pl.BlockSpec((1, 1, D), lambda e, f: (e, 0, 0)),
        ],
        out_specs=pl.BlockSpec((T, D), lambda e, f: (0, 0)),
        out_shape=jax.ShapeDtypeStruct((T, D), jnp.float32),
        scratch_shapes=[
            pltpu.VMEM((CAP, D), _BF),
            pltpu.VMEM((CAP, D), jnp.float32),
        ],
    )(xbf, mapc, slot, keep, gates, W1, b1r, W2, b2r)


def kernel(x, mu, cov_inv, W1, b1, W2, b2):
    gates, slot, keep, mapc, xbf = _route_plan(x, mu, cov_inv)
    return _moe(xbf, mapc, slot, keep, gates, W1, b1.reshape(E, 1, F),
                W2, b2.reshape(E, 1, D))
